# Initial kernel scaffold; baseline (speedup 1.0000x reference)
import jax
import jax.numpy as jnp
from jax.experimental import pallas as pl


def kernel(x, edge_index, W_proj, b_proj, W_l, b_l, W_r):
    raise NotImplementedError("write your pallas kernel here")


# speedup vs baseline: 3.5171x; 3.5171x over previous
import functools

import jax
import jax.numpy as jnp
from jax import lax
from jax.experimental import pallas as pl
from jax.experimental.pallas import tpu as pltpu
from jax.experimental.pallas import tpu_sc as plsc

N_NODES_K = 10000
DIM = 128
R_PAD = 10240
CH = 128
NCH = 80
CHUNK = 16
NW = 32
E_PAD = NW * NCH * CH
EROWS = E_PAD // CH
ROWS_PER_TILE = R_PAD // 16


def _project_tc(x, WpT, b_proj):
    M = x.shape[0]
    BM = 1000

    def body(x_ref, w_ref, b_ref, o_ref):
        o_ref[...] = jnp.maximum(
            jnp.dot(x_ref[...], w_ref[...], preferred_element_type=jnp.float32)
            + b_ref[...], 0.0)

    return pl.pallas_call(
        body,
        grid=(M // BM,),
        in_specs=[
            pl.BlockSpec((BM, DIM), lambda i: (i, 0)),
            pl.BlockSpec((DIM, DIM), lambda i: (0, 0)),
            pl.BlockSpec((1, DIM), lambda i: (0, 0)),
        ],
        out_specs=pl.BlockSpec((BM, DIM), lambda i: (i, 0)),
        out_shape=jax.ShapeDtypeStruct((M, DIM), jnp.float32),
    )(x, WpT, b_proj.reshape(1, DIM))


def _sc_aggregate(xp, src2, dst2):
    mesh = plsc.VectorSubcoreMesh(core_axis_name="c", subcore_axis_name="s")

    @functools.partial(
        pl.kernel,
        mesh=mesh,
        out_type=(
            jax.ShapeDtypeStruct((2 * R_PAD, DIM), jnp.float32),
            jax.ShapeDtypeStruct((2 * R_PAD, DIM), jnp.float32),
        ),
        scratch_types=[
            pltpu.VMEM((CHUNK,), jnp.int32),
            pltpu.VMEM((CHUNK, CH), jnp.int32),
            pltpu.VMEM((CHUNK, CH), jnp.int32),
            pltpu.VMEM((CH, DIM), jnp.float32),
            pltpu.VMEM_SHARED((R_PAD, DIM), jnp.float32),
            pltpu.SemaphoreType.DMA,
        ],
    )
    def k(xp_hbm, src_hbm, dst_hbm, acc_out, cnt_out,
          ridx, sidx, didx, rows, acc_sh, sem):
        cid = lax.axis_index("c")
        sid = lax.axis_index("s")
        wid = sid * 2 + cid
        iota16 = lax.iota(jnp.int32, 16)
        row0 = sid * ROWS_PER_TILE

        def fill(v):
            v16 = jnp.full((16,), v, jnp.float32)

            def body(i, _):
                for t in range(DIM // 16):
                    rows[i, pl.ds(t * 16, 16)] = v16
                return 0

            lax.fori_loop(0, CH, body, 0)

        def zinit(t, _):
            pltpu.sync_copy(rows, acc_sh.at[pl.ds(row0 + t * CH, CH)])
            return 0

        def drain(out):
            def body(t, _):
                r = row0 + t * CH
                pltpu.sync_copy(acc_sh.at[pl.ds(r, CH)], rows)
                pltpu.sync_copy(rows, out.at[pl.ds(cid * R_PAD + r, CH)])
                return 0

            lax.fori_loop(0, ROWS_PER_TILE // CH, body, 0)

        fill(0.0)
        lax.fori_loop(0, ROWS_PER_TILE // CH, zinit, 0)
        plsc.subcore_barrier()

        def chunk1(c, _):
            ridx[pl.ds(0, CHUNK)] = iota16 + wid * NCH + c * CHUNK
            pltpu.async_copy(src_hbm.at[ridx], sidx, sem).wait()
            pltpu.async_copy(dst_hbm.at[ridx], didx, sem).wait()

            def step(j, _):
                pltpu.async_copy(xp_hbm.at[sidx.at[j]], rows, sem).wait()
                pltpu.sync_copy(rows, acc_sh.at[didx.at[j]], add=True)
                return 0

            lax.fori_loop(0, CHUNK, step, 0)
            return 0

        lax.fori_loop(0, NCH // CHUNK, chunk1, 0)
        plsc.subcore_barrier()
        drain(acc_out)

        fill(0.0)
        lax.fori_loop(0, ROWS_PER_TILE // CH, zinit, 0)
        fill(1.0)
        plsc.subcore_barrier()

        def chunk2(c, _):
            ridx[pl.ds(0, CHUNK)] = iota16 + wid * NCH + c * CHUNK
            pltpu.async_copy(dst_hbm.at[ridx], didx, sem).wait()

            def step(j, _):
                pltpu.sync_copy(rows, acc_sh.at[didx.at[j]], add=True)
                return 0

            lax.fori_loop(0, CHUNK, step, 0)
            return 0

        lax.fori_loop(0, NCH // CHUNK, chunk2, 0)
        plsc.subcore_barrier()
        drain(cnt_out)

    return k(xp, src2, dst2)


def _finalize_tc(xpad, acc2, cnt2, WlT, WrT, b_l):
    BM = 1280

    def body(x_ref, a_ref, c_ref, wl_ref, wr_ref, b_ref, o_ref):
        acc = a_ref[0] + a_ref[1]
        cnt = c_ref[0, :, 0:1] + c_ref[1, :, 0:1]
        mean = acc / jnp.maximum(cnt, 1.0)
        out = (jnp.dot(mean, wl_ref[...], preferred_element_type=jnp.float32)
               + jnp.dot(x_ref[...], wr_ref[...], preferred_element_type=jnp.float32)
               + b_ref[...])
        nrm = jnp.sqrt(jnp.sum(out * out, axis=1, keepdims=True))
        out = out / jnp.maximum(nrm, 1e-12)
        o_ref[...] = jnp.where(out > 0, out, jnp.exp(jnp.minimum(out, 0.0)) - 1.0)

    return pl.pallas_call(
        body,
        grid=(R_PAD // BM,),
        in_specs=[
            pl.BlockSpec((BM, DIM), lambda i: (i, 0)),
            pl.BlockSpec((2, BM, DIM), lambda i: (0, i, 0)),
            pl.BlockSpec((2, BM, DIM), lambda i: (0, i, 0)),
            pl.BlockSpec((DIM, DIM), lambda i: (0, 0)),
            pl.BlockSpec((DIM, DIM), lambda i: (0, 0)),
            pl.BlockSpec((1, DIM), lambda i: (0, 0)),
        ],
        out_specs=pl.BlockSpec((BM, DIM), lambda i: (i, 0)),
        out_shape=jax.ShapeDtypeStruct((R_PAD, DIM), jnp.float32),
    )(xpad, acc2, cnt2, WlT, WrT, b_l.reshape(1, DIM))


@jax.jit
def kernel(x, edge_index, W_proj, b_proj, W_l, b_l, W_r):
    src = edge_index[0].astype(jnp.int32)
    dst = edge_index[1].astype(jnp.int32)
    pad = E_PAD - src.shape[0]
    src2 = jnp.concatenate([src, jnp.zeros((pad,), jnp.int32)]).reshape(-1, CH)
    dst2 = jnp.concatenate(
        [dst, jnp.full((pad,), N_NODES_K, jnp.int32)]).reshape(-1, CH)

    xp = _project_tc(x, W_proj.T, b_proj)
    acc2, cnt2 = _sc_aggregate(xp, src2, dst2)
    acc2 = acc2.reshape(2, R_PAD, DIM)
    cnt2 = cnt2.reshape(2, R_PAD, DIM)

    xpad = jnp.pad(x, ((0, R_PAD - N_NODES_K), (0, 0)))
    out = _finalize_tc(xpad, acc2, cnt2, W_l.T, W_r.T, b_l)
    return out[:N_NODES_K]
